# baseline (speedup 1.0000x reference)
import functools

import jax
import jax.numpy as jnp
from jax import lax
from jax.experimental import pallas as pl
from jax.experimental.pallas import tpu as pltpu
from jax.experimental.pallas import tpu_sc as plsc

NC, NS = 2, 16
NW = NC * NS
B, H, D = 4096, 50, 64
VP = 500000
BPW = B // NW
G = 2
CH = BPW // G
GH = 104
NBUF = 4
LANES = 16
LG = D // LANES

_mesh = plsc.VectorSubcoreMesh(core_axis_name="c", subcore_axis_name="s",
                               num_cores=NC, num_subcores=NS)


@functools.partial(
    pl.kernel,
    out_type=jax.ShapeDtypeStruct((B, D), jnp.float32),
    mesh=_mesh,
    compiler_params=pltpu.CompilerParams(use_tc_tiling_on_sc=True),
    scratch_types=[
        pltpu.VMEM((CH, GH), jnp.int32),
        pltpu.VMEM((CH, GH), jnp.int32),
        [pltpu.VMEM((GH, 2 * D), jnp.float32) for _ in range(NBUF)],
        pltpu.VMEM((BPW, D), jnp.float32),
        [pltpu.SemaphoreType.DMA for _ in range(NBUF)],
    ],
)
def _sc_embed_sum(pidx_hbm, offs_hbm, table_hbm, out_hbm, pidx_v, offs_v,
                  bufs, out_v, sems):
    wid = lax.axis_index("s") * NC + lax.axis_index("c")
    pltpu.sync_copy(pidx_hbm.at[wid], pidx_v)
    pltpu.sync_copy(offs_hbm.at[wid], offs_v)

    def start(g, b):
        pltpu.async_copy(table_hbm.at[pidx_v.at[g]], bufs[b], sems[b])

    def wait(b):
        pltpu.make_async_copy(table_hbm.at[pl.ds(0, GH)], bufs[b],
                              sems[b]).wait()

    def accum(buf, g):
        starts = [0, 16, 32, 48, 64, 80, 88]
        ovecs = [offs_v[g, pl.ds(s, LANES)] for s in starts]

        def off_at(row):
            if row < 96:
                return ovecs[row // 16][row % 16]
            return ovecs[6][row - 88]

        for e in range(G):
            accs = None
            for r in range(H):
                row = e * H + r
                o = pl.multiple_of(off_at(row), D)
                vals = [buf[row, pl.ds(o + l * LANES, LANES)]
                        for l in range(LG)]
                if accs is None:
                    accs = vals
                else:
                    accs = [a + v for a, v in zip(accs, vals)]
            for l in range(LG):
                out_v[g * G + e, pl.ds(l * LANES, LANES)] = accs[l]

    for b in range(NBUF - 1):
        start(b, b)

    def body(i, carry):
        g0 = NBUF * i
        for b in range(NBUF):
            g = g0 + b

            nb = (b + NBUF - 1) % NBUF

            @pl.when(g + NBUF - 1 < CH)
            def _():
                start(g + NBUF - 1, nb)

            wait(b)
            accum(bufs[b], g)
        return carry

    lax.fori_loop(0, CH // NBUF, body, 0)
    pltpu.sync_copy(out_v, out_hbm.at[pl.ds(wid * BPW, BPW)])


def kernel(query, table):
    q = query.reshape(NW, CH, G * H)
    q = jnp.pad(q, ((0, 0), (0, 0), (0, GH - G * H)))
    pidx = jax.lax.shift_right_logical(q, 1)
    offs = jax.lax.shift_left(jnp.bitwise_and(q, 1), 6)
    t2 = table.reshape(VP, 2 * D)
    out = _sc_embed_sum(pidx, offs, t2)
    return out[:, None, :]
